# Initial kernel scaffold; baseline (speedup 1.0000x reference)
import jax
import jax.numpy as jnp
from jax.experimental import pallas as pl


def kernel(tags, keypoint_indices):
    raise NotImplementedError("write your pallas kernel here")


# speedup vs baseline: 4.0530x; 4.0530x over previous
import functools

import jax
import jax.numpy as jnp
from jax import lax
from jax.experimental import pallas as pl
from jax.experimental.pallas import tpu as pltpu
from jax.experimental.pallas import tpu_sc as plsc

_B, _N, _K, _L, _H, _W = 4, 32, 17, 4, 256, 256
_HW = _H * _W
_I = _B * _N
_G = _I // 16
_GPB = _N // 16
_NIDX = _K * _L * _I
_NCH = _NIDX // 128
_LOSS_WEIGHT = 1.0
_PUSH_FACTOR = 0.5
_EPS = 1e-6


def _sc_body(tags_hbm, kpi_hbm, out_hbm,
             kpi_v, idx_v, kt_v, m_v, tag_v, valid_v, pullc_v, out_v, sem):
    wid = lax.axis_index("s") * 2 + lax.axis_index("c")

    @pl.when(wid == 0)
    def _():
        pltpu.sync_copy(kpi_hbm, kpi_v)

        lane = lax.iota(jnp.int32, 16)

        @pl.loop(0, _K)
        def _idx_build(k):
            for g in range(_G):
                b = g // _GPB
                base = (lane + 16 * g) * (_K * 2) + 2 * k
                idxv = plsc.load_gather(kpi_v, [base])
                mv = plsc.load_gather(kpi_v, [base + 1])
                m_v[k, pl.ds(16 * g, 16)] = jnp.where(
                    mv != 0, jnp.float32(1.0), jnp.float32(0.0))
                for l in range(_L):
                    off = (b * _L + l) * _K * _HW + k * _HW
                    idx_v[k * _L + l, pl.ds(16 * g, 16)] = idxv + off

        @pl.loop(0, _NCH)
        def _fire(c):
            pltpu.make_async_copy(
                tags_hbm.at[idx_v.at[c]], kt_v.at[c], sem).start()

        @pl.loop(0, _NCH)
        def _drain(c):
            pltpu.make_async_copy(
                tags_hbm.at[idx_v.at[c]], kt_v.at[c], sem).wait()

        for g in range(_G):
            sl = pl.ds(16 * g, 16)

            def _sums(k, carry):
                cnt, t0, t1, t2, t3 = carry
                mv = m_v[k, sl]
                cnt = cnt + mv
                t0 = t0 + mv * kt_v[k * _L + 0, sl]
                t1 = t1 + mv * kt_v[k * _L + 1, sl]
                t2 = t2 + mv * kt_v[k * _L + 2, sl]
                t3 = t3 + mv * kt_v[k * _L + 3, sl]
                return cnt, t0, t1, t2, t3

            z = jnp.zeros((16,), jnp.float32)
            cnt, t0, t1, t2, t3 = lax.fori_loop(
                0, _K, _sums, (z, z, z, z, z))
            safe = jnp.maximum(cnt, 1.0)
            tg = [t0 / safe, t1 / safe, t2 / safe, t3 / safe]
            for l in range(_L):
                tag_v[l, sl] = tg[l]
            valid = jnp.where(cnt > 0.0, jnp.float32(1.0), jnp.float32(0.0))
            valid_v[sl] = valid

            def _pull(k, acc):
                mv = m_v[k, sl]
                s = jnp.zeros((16,), jnp.float32)
                for l in range(_L):
                    d = kt_v[k * _L + l, sl] - tg[l]
                    s = s + d * d
                return acc + mv * s

            psum = lax.fori_loop(0, _K, _pull, z)
            pullc_v[sl] = valid * psum / (safe * jnp.float32(_L))

        zf = jnp.zeros((16,), jnp.float32)
        total_pull = zf
        total_push = zf
        for b in range(_B):
            g0, g1 = _GPB * b, _GPB * b + 1
            s0, s1 = pl.ds(16 * g0, 16), pl.ds(16 * g1, 16)
            v0, v1 = valid_v[s0], valid_v[s1]
            nn = zf + (jnp.sum(v0) + jnp.sum(v1))
            pull_img = (zf + (jnp.sum(pullc_v[s0]) + jnp.sum(pullc_v[s1]))) \
                / (nn + jnp.float32(_EPS))

            tj = [[tag_v[l, s0] for l in range(_L)],
                  [tag_v[l, s1] for l in range(_L)]]

            def _push(i, acc):
                idxv = jnp.zeros((16,), jnp.int32) + (_N * b + i)
                vi = plsc.load_gather(valid_v, [idxv])
                e0 = jnp.zeros((16,), jnp.float32)
                e1 = jnp.zeros((16,), jnp.float32)
                for l in range(_L):
                    lv = jnp.zeros((16,), jnp.int32) + l
                    ti = plsc.load_gather(tag_v, [lv, idxv])
                    d0 = tj[0][l] - ti
                    d1 = tj[1][l] - ti
                    e0 = e0 + jnp.exp(-(d0 * d0))
                    e1 = e1 + jnp.exp(-(d1 * d1))
                return acc + vi * (v0 * e0 + v1 * e1)

            pacc = lax.fori_loop(0, _N, _push,
                                 jnp.zeros((16,), jnp.float32))
            push_sum = zf + jnp.sum(pacc)
            push_img = jnp.where(
                nn > 1.0,
                push_sum / ((nn - 1.0) * nn + jnp.float32(_EPS)),
                zf)
            total_pull = total_pull + pull_img
            total_push = total_push + push_img

        lane_f = lax.iota(jnp.int32, 16)
        out_v[...] = (jnp.where(lane_f == 0, total_pull, zf)
                      + jnp.where(lane_f == 1, total_push, zf))
        pltpu.sync_copy(out_v, out_hbm)


@functools.partial(
    pl.kernel,
    out_type=jax.ShapeDtypeStruct((16,), jnp.float32),
    mesh=plsc.VectorSubcoreMesh(core_axis_name="c", subcore_axis_name="s"),
    compiler_params=pltpu.CompilerParams(needs_layout_passes=False),
    scratch_types=[
        pltpu.VMEM((_I * _K * 2,), jnp.int32),
        pltpu.VMEM((_NCH, 128), jnp.int32),
        pltpu.VMEM((_NCH, 128), jnp.float32),
        pltpu.VMEM((_K, 128), jnp.float32),
        pltpu.VMEM((_L, 128), jnp.float32),
        pltpu.VMEM((128,), jnp.float32),
        pltpu.VMEM((128,), jnp.float32),
        pltpu.VMEM((16,), jnp.float32),
        pltpu.SemaphoreType.DMA,
    ],
)
def _ae_loss_sc(tags_hbm, kpi_hbm, out_hbm, *scratch):
    _sc_body(tags_hbm, kpi_hbm, out_hbm, *scratch)


@jax.jit
def kernel(tags, keypoint_indices):
    tags_flat = tags.reshape(-1)
    kpi_flat = keypoint_indices.astype(jnp.int32).reshape(-1)
    out = _ae_loss_sc(tags_flat, kpi_flat)
    pull_loss = out[0] * jnp.float32(_LOSS_WEIGHT)
    push_loss = out[1] * jnp.float32(_LOSS_WEIGHT * _PUSH_FACTOR)
    return (pull_loss, push_loss)


# speedup vs baseline: 9.3886x; 2.3165x over previous
import functools

import jax
import jax.numpy as jnp
from jax import lax
from jax.experimental import pallas as pl
from jax.experimental.pallas import tpu as pltpu
from jax.experimental.pallas import tpu_sc as plsc

_B, _N, _K, _L, _H, _W = 4, 32, 17, 4, 256, 256
_HW = _H * _W
_I = _B * _N
_G = _I // 16
_GPB = _N // 16
_NIDX = _K * _L * _I
_NCH = _NIDX // 128
_LOSS_WEIGHT = 1.0
_PUSH_FACTOR = 0.5
_EPS = 1e-6


def _sc_body(tags_hbm, kpi_hbm, out_hbm,
             kpi_v, idx_v, kt_v, m_v, tag_v, valid_v, pullc_v, out_v, sem):
    wid = lax.axis_index("s") * 2 + lax.axis_index("c")

    @pl.when(wid == 0)
    def _():
        pltpu.sync_copy(kpi_hbm, kpi_v)

        lane = lax.iota(jnp.int32, 16)

        @pl.loop(0, _K)
        def _idx_build(k):
            for g in range(_G):
                b = g // _GPB
                base = (lane + 16 * g) * (_K * 2) + 2 * k
                idxv = plsc.load_gather(kpi_v, [base])
                mv = plsc.load_gather(kpi_v, [base + 1])
                m_v[k, pl.ds(16 * g, 16)] = jnp.where(
                    mv != 0, jnp.float32(1.0), jnp.float32(0.0))
                pv = (((idxv >> 11) << 11)
                      + (((idxv >> 7) & 1) << 10)
                      + (((idxv >> 8) & 7) << 7)
                      + (idxv & 127))
                for l in range(_L):
                    off = (b * _L + l) * _K * _HW + k * _HW
                    idx_v[k * _L + l, pl.ds(16 * g, 16)] = pv + off

        @pl.loop(0, _NCH)
        def _fire(c):
            pltpu.make_async_copy(
                tags_hbm.at[idx_v.at[c]], kt_v.at[c], sem).start()

        @pl.loop(0, _NCH)
        def _drain(c):
            pltpu.make_async_copy(
                tags_hbm.at[idx_v.at[c]], kt_v.at[c], sem).wait()

        for g in range(_G):
            sl = pl.ds(16 * g, 16)

            def _sums(k, carry):
                cnt, t0, t1, t2, t3 = carry
                mv = m_v[k, sl]
                cnt = cnt + mv
                t0 = t0 + mv * kt_v[k * _L + 0, sl]
                t1 = t1 + mv * kt_v[k * _L + 1, sl]
                t2 = t2 + mv * kt_v[k * _L + 2, sl]
                t3 = t3 + mv * kt_v[k * _L + 3, sl]
                return cnt, t0, t1, t2, t3

            z = jnp.zeros((16,), jnp.float32)
            cnt, t0, t1, t2, t3 = lax.fori_loop(
                0, _K, _sums, (z, z, z, z, z))
            safe = jnp.maximum(cnt, 1.0)
            tg = [t0 / safe, t1 / safe, t2 / safe, t3 / safe]
            for l in range(_L):
                tag_v[l, sl] = tg[l]
            valid = jnp.where(cnt > 0.0, jnp.float32(1.0), jnp.float32(0.0))
            valid_v[sl] = valid

            def _pull(k, acc):
                mv = m_v[k, sl]
                s = jnp.zeros((16,), jnp.float32)
                for l in range(_L):
                    d = kt_v[k * _L + l, sl] - tg[l]
                    s = s + d * d
                return acc + mv * s

            psum = lax.fori_loop(0, _K, _pull, z)
            pullc_v[sl] = valid * psum / (safe * jnp.float32(_L))

        zf = jnp.zeros((16,), jnp.float32)
        total_pull = zf
        total_push = zf
        for b in range(_B):
            g0, g1 = _GPB * b, _GPB * b + 1
            s0, s1 = pl.ds(16 * g0, 16), pl.ds(16 * g1, 16)
            v0, v1 = valid_v[s0], valid_v[s1]
            nn = zf + (jnp.sum(v0) + jnp.sum(v1))
            pull_img = (zf + (jnp.sum(pullc_v[s0]) + jnp.sum(pullc_v[s1]))) \
                / (nn + jnp.float32(_EPS))

            tj = [[tag_v[l, s0] for l in range(_L)],
                  [tag_v[l, s1] for l in range(_L)]]

            def _push(i, acc):
                idxv = jnp.zeros((16,), jnp.int32) + (_N * b + i)
                vi = plsc.load_gather(valid_v, [idxv])
                e0 = jnp.zeros((16,), jnp.float32)
                e1 = jnp.zeros((16,), jnp.float32)
                for l in range(_L):
                    lv = jnp.zeros((16,), jnp.int32) + l
                    ti = plsc.load_gather(tag_v, [lv, idxv])
                    d0 = tj[0][l] - ti
                    d1 = tj[1][l] - ti
                    e0 = e0 + jnp.exp(-(d0 * d0))
                    e1 = e1 + jnp.exp(-(d1 * d1))
                return acc + vi * (v0 * e0 + v1 * e1)

            pacc = lax.fori_loop(0, _N, _push,
                                 jnp.zeros((16,), jnp.float32))
            push_sum = zf + jnp.sum(pacc)
            push_img = jnp.where(
                nn > 1.0,
                push_sum / ((nn - 1.0) * nn + jnp.float32(_EPS)),
                zf)
            total_pull = total_pull + pull_img
            total_push = total_push + push_img

        lane_f = lax.iota(jnp.int32, 16)
        out_v[...] = (jnp.where(lane_f == 0, total_pull, zf)
                      + jnp.where(lane_f == 1, total_push, zf))
        pltpu.sync_copy(out_v, out_hbm)


@functools.partial(
    pl.kernel,
    out_type=jax.ShapeDtypeStruct((16,), jnp.float32),
    mesh=plsc.VectorSubcoreMesh(core_axis_name="c", subcore_axis_name="s"),
    compiler_params=pltpu.CompilerParams(needs_layout_passes=False),
    scratch_types=[
        pltpu.VMEM((_I * _K * 2,), jnp.int32),
        pltpu.VMEM((_NCH, 128), jnp.int32),
        pltpu.VMEM((_NCH, 128), jnp.float32),
        pltpu.VMEM((_K, 128), jnp.float32),
        pltpu.VMEM((_L, 128), jnp.float32),
        pltpu.VMEM((128,), jnp.float32),
        pltpu.VMEM((128,), jnp.float32),
        pltpu.VMEM((16,), jnp.float32),
        pltpu.SemaphoreType.DMA,
    ],
)
def _ae_loss_sc(tags_hbm, kpi_hbm, out_hbm, *scratch):
    _sc_body(tags_hbm, kpi_hbm, out_hbm, *scratch)


@jax.jit
def kernel(tags, keypoint_indices):
    tags_flat = (tags
                 .reshape(_B, _L * _K, _H // 8, 8, _W // 128, 128)
                 .transpose(0, 1, 2, 4, 3, 5)
                 .reshape(-1))
    kpi_flat = keypoint_indices.astype(jnp.int32).reshape(-1)
    out = _ae_loss_sc(tags_flat, kpi_flat)
    pull_loss = out[0] * jnp.float32(_LOSS_WEIGHT)
    push_loss = out[1] * jnp.float32(_LOSS_WEIGHT * _PUSH_FACTOR)
    return (pull_loss, push_loss)


# speedup vs baseline: 9.7986x; 1.0437x over previous
import functools

import jax
import jax.numpy as jnp
from jax import lax
from jax.experimental import pallas as pl
from jax.experimental.pallas import tpu as pltpu
from jax.experimental.pallas import tpu_sc as plsc

_B, _N, _K, _L, _H, _W = 4, 32, 17, 4, 256, 256
_HW = _H * _W
_I = _B * _N
_G = _I // 16
_GPB = _N // 16
_NIDX = _K * _L * _I
_NCH = _NIDX // 128
_LOSS_WEIGHT = 1.0
_PUSH_FACTOR = 0.5
_EPS = 1e-6


def _sc_body(tags_hbm, kpi_hbm, out_hbm,
             kpi_v, idx_v, kt_v, m_v, tag_v, valid_v, pullc_v, out_v, sem):
    wid = lax.axis_index("s") * 2 + lax.axis_index("c")

    @pl.when(wid == 0)
    def _():
        pltpu.sync_copy(kpi_hbm, kpi_v)

        lane = lax.iota(jnp.int32, 16)

        @pl.loop(0, _K)
        def _idx_build(k):
            for g in range(_G):
                b = g // _GPB
                base = (lane + 16 * g) * (_K * 2) + 2 * k
                idxv = plsc.load_gather(kpi_v, [base])
                mv = plsc.load_gather(kpi_v, [base + 1])
                m_v[k, pl.ds(16 * g, 16)] = jnp.where(
                    mv != 0, jnp.float32(1.0), jnp.float32(0.0))
                pv = (((idxv >> 11) << 11)
                      + (((idxv >> 7) & 1) << 10)
                      + (((idxv >> 8) & 7) << 7)
                      + (idxv & 127))
                for l in range(_L):
                    off = (b * _L + l) * _K * _HW + k * _HW
                    idx_v[k * _L + l, pl.ds(16 * g, 16)] = pv + off

        @pl.loop(0, _NCH)
        def _fire(c):
            pltpu.make_async_copy(
                tags_hbm.at[idx_v.at[c]], kt_v.at[c], sem).start()

        @pl.loop(0, _NCH)
        def _drain(c):
            pltpu.make_async_copy(
                tags_hbm.at[idx_v.at[c]], kt_v.at[c], sem).wait()

        for g in range(_G):
            sl = pl.ds(16 * g, 16)

            def _sums(k, carry):
                cnt, t0, t1, t2, t3 = carry
                mv = m_v[k, sl]
                cnt = cnt + mv
                t0 = t0 + mv * kt_v[k * _L + 0, sl]
                t1 = t1 + mv * kt_v[k * _L + 1, sl]
                t2 = t2 + mv * kt_v[k * _L + 2, sl]
                t3 = t3 + mv * kt_v[k * _L + 3, sl]
                return cnt, t0, t1, t2, t3

            z = jnp.zeros((16,), jnp.float32)
            cnt, t0, t1, t2, t3 = lax.fori_loop(
                0, _K, _sums, (z, z, z, z, z))
            safe = jnp.maximum(cnt, 1.0)
            tg = [t0 / safe, t1 / safe, t2 / safe, t3 / safe]
            for l in range(_L):
                tag_v[l, sl] = tg[l]
            valid = jnp.where(cnt > 0.0, jnp.float32(1.0), jnp.float32(0.0))
            valid_v[sl] = valid

            def _pull(k, acc):
                mv = m_v[k, sl]
                s = jnp.zeros((16,), jnp.float32)
                for l in range(_L):
                    d = kt_v[k * _L + l, sl] - tg[l]
                    s = s + d * d
                return acc + mv * s

            psum = lax.fori_loop(0, _K, _pull, z)
            pullc_v[sl] = valid * psum / (safe * jnp.float32(_L))

        zf = jnp.zeros((16,), jnp.float32)
        total_pull = zf
        total_push = zf
        for b in range(_B):
            g0, g1 = _GPB * b, _GPB * b + 1
            s0, s1 = pl.ds(16 * g0, 16), pl.ds(16 * g1, 16)
            v0, v1 = valid_v[s0], valid_v[s1]
            nn = zf + (jnp.sum(v0) + jnp.sum(v1))
            pull_img = (zf + (jnp.sum(pullc_v[s0]) + jnp.sum(pullc_v[s1]))) \
                / (nn + jnp.float32(_EPS))

            tj = [[tag_v[l, s0] for l in range(_L)],
                  [tag_v[l, s1] for l in range(_L)]]

            def _push(i, acc):
                idxv = jnp.zeros((16,), jnp.int32) + (_N * b + i)
                vi = plsc.load_gather(valid_v, [idxv])
                e0 = jnp.zeros((16,), jnp.float32)
                e1 = jnp.zeros((16,), jnp.float32)
                for l in range(_L):
                    lv = jnp.zeros((16,), jnp.int32) + l
                    ti = plsc.load_gather(tag_v, [lv, idxv])
                    d0 = tj[0][l] - ti
                    d1 = tj[1][l] - ti
                    e0 = e0 + jnp.exp(-(d0 * d0))
                    e1 = e1 + jnp.exp(-(d1 * d1))
                return acc + vi * (v0 * e0 + v1 * e1)

            pacc = lax.fori_loop(0, _N, _push,
                                 jnp.zeros((16,), jnp.float32))
            push_sum = zf + jnp.sum(pacc)
            push_img = jnp.where(
                nn > 1.0,
                push_sum / ((nn - 1.0) * nn + jnp.float32(_EPS)),
                zf)
            total_pull = total_pull + pull_img
            total_push = total_push + push_img

        lane_f = lax.iota(jnp.int32, 16)
        out_v[...] = (jnp.where(lane_f == 0, total_pull, zf)
                      + jnp.where(lane_f == 1, total_push, zf))
        pltpu.sync_copy(out_v, out_hbm)


@functools.partial(
    pl.kernel,
    out_type=jax.ShapeDtypeStruct((16,), jnp.float32),
    mesh=plsc.VectorSubcoreMesh(core_axis_name="c", subcore_axis_name="s", num_cores=1),
    compiler_params=pltpu.CompilerParams(needs_layout_passes=False),
    scratch_types=[
        pltpu.VMEM((_I * _K * 2,), jnp.int32),
        pltpu.VMEM((_NCH, 128), jnp.int32),
        pltpu.VMEM((_NCH, 128), jnp.float32),
        pltpu.VMEM((_K, 128), jnp.float32),
        pltpu.VMEM((_L, 128), jnp.float32),
        pltpu.VMEM((128,), jnp.float32),
        pltpu.VMEM((128,), jnp.float32),
        pltpu.VMEM((16,), jnp.float32),
        pltpu.SemaphoreType.DMA,
    ],
)
def _ae_loss_sc(tags_hbm, kpi_hbm, out_hbm, *scratch):
    _sc_body(tags_hbm, kpi_hbm, out_hbm, *scratch)


@jax.jit
def kernel(tags, keypoint_indices):
    tags_flat = (tags
                 .reshape(_B, _L * _K, _H // 8, 8, _W // 128, 128)
                 .transpose(0, 1, 2, 4, 3, 5)
                 .reshape(-1))
    kpi_flat = keypoint_indices.astype(jnp.int32).reshape(-1)
    out = _ae_loss_sc(tags_flat, kpi_flat)
    pull_loss = out[0] * jnp.float32(_LOSS_WEIGHT)
    push_loss = out[1] * jnp.float32(_LOSS_WEIGHT * _PUSH_FACTOR)
    return (pull_loss, push_loss)


# speedup vs baseline: 13.4582x; 1.3735x over previous
import functools

import jax
import jax.numpy as jnp
from jax import lax
from jax.experimental import pallas as pl
from jax.experimental.pallas import tpu as pltpu
from jax.experimental.pallas import tpu_sc as plsc

_B, _N, _K, _L, _H, _W = 4, 32, 17, 4, 256, 256
_HW = _H * _W
_I = _B * _N
_G = _I // 16
_GPB = _N // 16
_EPG = _K * _L * 16
_NCH = 9
_LOSS_WEIGHT = 1.0
_PUSH_FACTOR = 0.5
_EPS = 1e-6


def _sc_body(tags_hbm, kpi_hbm, out_hbm,
             kpi_loc, idx_loc, kt_loc, m_loc, stats_loc, other_loc,
             pacc_buf, all_stats, all_push, out_v,
             shared_stats, shared_push, sem):
    g = lax.axis_index("s")
    zf = jnp.zeros((16,), jnp.float32)
    zi = jnp.zeros((16,), jnp.int32)
    lane = lax.iota(jnp.int32, 16)

    @pl.when(g < _G)
    def _():
        pltpu.sync_copy(kpi_hbm, kpi_loc)
        b = g // _GPB

        for c in range(4):
            idx_loc[_NCH - 1, pl.ds(64 + 16 * c, 16)] = zi

        @pl.loop(0, _K)
        def _idx_build(k):
            base = (lane + 16 * g) * (_K * 2) + 2 * k
            idxv = plsc.load_gather(kpi_loc, [base])
            mv = plsc.load_gather(kpi_loc, [base + 1])
            m_loc[k, :] = jnp.where(mv != 0, jnp.float32(1.0),
                                    jnp.float32(0.0))
            pv = (((idxv >> 11) << 11)
                  + (((idxv >> 7) & 1) << 10)
                  + (((idxv >> 8) & 7) << 7)
                  + (idxv & 127))
            coff = (b * _L) * _K * _HW + k * _HW
            for l in range(_L):
                e = 4 * k + l
                idx_loc[e >> 3, pl.ds((e & 7) * 16, 16)] = \
                    pv + (coff + l * _K * _HW)

        for c in range(_NCH):
            pltpu.make_async_copy(
                tags_hbm.at[idx_loc.at[c]], kt_loc.at[c], sem).start()
        for c in range(_NCH):
            pltpu.make_async_copy(
                tags_hbm.at[idx_loc.at[c]], kt_loc.at[c], sem).wait()

        def _kt(k, l):
            e = 4 * k + l
            return kt_loc[e >> 3, pl.ds((e & 7) * 16, 16)]

        def _sums(k, carry):
            cnt, t0, t1, t2, t3 = carry
            mv = m_loc[k, :]
            return (cnt + mv,
                    t0 + mv * _kt(k, 0), t1 + mv * _kt(k, 1),
                    t2 + mv * _kt(k, 2), t3 + mv * _kt(k, 3))

        cnt, t0, t1, t2, t3 = lax.fori_loop(
            0, _K, _sums, (zf, zf, zf, zf, zf))
        safe = jnp.maximum(cnt, 1.0)
        tg = [t0 / safe, t1 / safe, t2 / safe, t3 / safe]
        valid = jnp.where(cnt > 0.0, jnp.float32(1.0), jnp.float32(0.0))

        def _pull(k, acc):
            mv = m_loc[k, :]
            s = zf
            for l in range(_L):
                d = _kt(k, l) - tg[l]
                s = s + d * d
            return acc + mv * s

        psum = lax.fori_loop(0, _K, _pull, zf)
        for l in range(_L):
            stats_loc[pl.ds(16 * l, 16)] = tg[l]
        stats_loc[pl.ds(16 * _L, 16)] = valid
        stats_loc[pl.ds(16 * (_L + 1), 16)] = \
            valid * psum / (safe * jnp.float32(_L))
        pltpu.sync_copy(stats_loc, shared_stats.at[g])

    plsc.subcore_barrier()

    @pl.when(g < _G)
    def _():
        pltpu.sync_copy(shared_stats.at[g ^ 1], other_loc)
        v_own = stats_loc[pl.ds(16 * _L, 16)]
        v_oth = other_loc[pl.ds(16 * _L, 16)]
        t_own = [stats_loc[pl.ds(16 * l, 16)] for l in range(_L)]
        t_oth = [other_loc[pl.ds(16 * l, 16)] for l in range(_L)]

        def _push(i, acc):
            iv = zi + i
            vi = plsc.load_gather(stats_loc, [iv + 16 * _L])
            e_own = zf
            e_oth = zf
            for l in range(_L):
                ti = plsc.load_gather(stats_loc, [iv + 16 * l])
                d0 = t_own[l] - ti
                d1 = t_oth[l] - ti
                e_own = e_own + jnp.exp(-(d0 * d0))
                e_oth = e_oth + jnp.exp(-(d1 * d1))
            return acc + vi * (v_own * e_own + v_oth * e_oth)

        pacc_buf[pl.ds(0, 16)] = lax.fori_loop(0, 16, _push, zf)
        pltpu.sync_copy(pacc_buf, shared_push.at[g])

    plsc.subcore_barrier()

    @pl.when(g == 0)
    def _():
        pltpu.sync_copy(shared_stats, all_stats)
        pltpu.sync_copy(shared_push, all_push)
        total_pull = zf
        total_push = zf
        for b in range(_B):
            g0, g1 = _GPB * b, _GPB * b + 1
            va = all_stats[g0, pl.ds(16 * _L, 16)]
            vb = all_stats[g1, pl.ds(16 * _L, 16)]
            nn = zf + (jnp.sum(va) + jnp.sum(vb))
            pull_img = (zf + (jnp.sum(all_stats[g0, pl.ds(16 * (_L + 1), 16)])
                              + jnp.sum(all_stats[g1, pl.ds(16 * (_L + 1), 16)]))) \
                / (nn + jnp.float32(_EPS))
            push_sum = zf + (jnp.sum(all_push[g0, pl.ds(0, 16)])
                             + jnp.sum(all_push[g1, pl.ds(0, 16)]))
            push_img = jnp.where(
                nn > 1.0,
                push_sum / ((nn - 1.0) * nn + jnp.float32(_EPS)),
                zf)
            total_pull = total_pull + pull_img
            total_push = total_push + push_img
        out_v[...] = (jnp.where(lane == 0, total_pull, zf)
                      + jnp.where(lane == 1, total_push, zf))
        pltpu.sync_copy(out_v, out_hbm)


@functools.partial(
    pl.kernel,
    out_type=jax.ShapeDtypeStruct((16,), jnp.float32),
    mesh=plsc.VectorSubcoreMesh(core_axis_name="c", subcore_axis_name="s",
                                num_cores=1),
    compiler_params=pltpu.CompilerParams(needs_layout_passes=False),
    scratch_types=[
        pltpu.VMEM((_I * _K * 2,), jnp.int32),
        pltpu.VMEM((_NCH, 128), jnp.int32),
        pltpu.VMEM((_NCH, 128), jnp.float32),
        pltpu.VMEM((_K, 16), jnp.float32),
        pltpu.VMEM((128,), jnp.float32),
        pltpu.VMEM((128,), jnp.float32),
        pltpu.VMEM((128,), jnp.float32),
        pltpu.VMEM((_G, 128), jnp.float32),
        pltpu.VMEM((_G, 128), jnp.float32),
        pltpu.VMEM((16,), jnp.float32),
        pltpu.VMEM_SHARED((_G, 128), jnp.float32),
        pltpu.VMEM_SHARED((_G, 128), jnp.float32),
        pltpu.SemaphoreType.DMA,
    ],
)
def _ae_loss_sc(tags_hbm, kpi_hbm, out_hbm, *scratch):
    _sc_body(tags_hbm, kpi_hbm, out_hbm, *scratch)


@jax.jit
def kernel(tags, keypoint_indices):
    tags_flat = (tags
                 .reshape(_B, _L * _K, _H // 8, 8, _W // 128, 128)
                 .transpose(0, 1, 2, 4, 3, 5)
                 .reshape(-1))
    kpi_flat = keypoint_indices.astype(jnp.int32).reshape(-1)
    out = _ae_loss_sc(tags_flat, kpi_flat)
    pull_loss = out[0] * jnp.float32(_LOSS_WEIGHT)
    push_loss = out[1] * jnp.float32(_LOSS_WEIGHT * _PUSH_FACTOR)
    return (pull_loss, push_loss)


# speedup vs baseline: 13.6306x; 1.0128x over previous
import functools

import jax
import jax.numpy as jnp
from jax import lax
from jax.experimental import pallas as pl
from jax.experimental.pallas import tpu as pltpu
from jax.experimental.pallas import tpu_sc as plsc

_B, _N, _K, _L, _H, _W = 4, 32, 17, 4, 256, 256
_HW = _H * _W
_I = _B * _N
_G = _I // 16
_GPB = _N // 16
_EPG = _K * _L * 16
_NCH = 9
_LOSS_WEIGHT = 1.0
_PUSH_FACTOR = 0.5
_EPS = 1e-6


def _sc_body(tags_hbm, kpi_hbm, out_hbm,
             kpi_loc, idx_loc, kt_loc, m_loc, stats_loc, other_loc,
             pacc_buf, all_stats, all_push, out_v,
             shared_stats, shared_push, sem):
    g = lax.axis_index("s")
    zf = jnp.zeros((16,), jnp.float32)
    zi = jnp.zeros((16,), jnp.int32)
    lane = lax.iota(jnp.int32, 16)

    @pl.when(g < _G)
    def _():
        pltpu.sync_copy(kpi_hbm.at[pl.ds(g * (_K * 2 * 16), _K * 2 * 16)],
                        kpi_loc)
        b = g // _GPB

        for c in range(4):
            idx_loc[_NCH - 1, pl.ds(64 + 16 * c, 16)] = zi

        for k in range(_K):
            base = lane * (_K * 2) + 2 * k
            idxv = plsc.load_gather(kpi_loc, [base])
            mv = plsc.load_gather(kpi_loc, [base + 1])
            m_loc[k, :] = jnp.where(mv != 0, jnp.float32(1.0),
                                    jnp.float32(0.0))
            pv = (((idxv >> 11) << 11)
                  + (((idxv >> 7) & 1) << 10)
                  + (((idxv >> 8) & 7) << 7)
                  + (idxv & 127))
            coff = (b * _L) * _K * _HW + k * _HW
            for l in range(_L):
                e = 4 * k + l
                idx_loc[e >> 3, pl.ds((e & 7) * 16, 16)] = \
                    pv + (coff + l * _K * _HW)
            if k % 2 == 1:
                c = (4 * k + 3) // 8
                pltpu.make_async_copy(
                    tags_hbm.at[idx_loc.at[c]], kt_loc.at[c], sem).start()
        pltpu.make_async_copy(
            tags_hbm.at[idx_loc.at[_NCH - 1]], kt_loc.at[_NCH - 1],
            sem).start()
        for c in range(_NCH):
            pltpu.make_async_copy(
                tags_hbm.at[idx_loc.at[c]], kt_loc.at[c], sem).wait()

        def _kt(k, l):
            e = 4 * k + l
            return kt_loc[e >> 3, pl.ds((e & 7) * 16, 16)]

        def _sums(k, carry):
            cnt, t0, t1, t2, t3 = carry
            mv = m_loc[k, :]
            return (cnt + mv,
                    t0 + mv * _kt(k, 0), t1 + mv * _kt(k, 1),
                    t2 + mv * _kt(k, 2), t3 + mv * _kt(k, 3))

        cnt, t0, t1, t2, t3 = lax.fori_loop(
            0, _K, _sums, (zf, zf, zf, zf, zf))
        safe = jnp.maximum(cnt, 1.0)
        tg = [t0 / safe, t1 / safe, t2 / safe, t3 / safe]
        valid = jnp.where(cnt > 0.0, jnp.float32(1.0), jnp.float32(0.0))

        def _pull(k, acc):
            mv = m_loc[k, :]
            s = zf
            for l in range(_L):
                d = _kt(k, l) - tg[l]
                s = s + d * d
            return acc + mv * s

        psum = lax.fori_loop(0, _K, _pull, zf)
        for l in range(_L):
            stats_loc[pl.ds(16 * l, 16)] = tg[l]
        stats_loc[pl.ds(16 * _L, 16)] = valid
        stats_loc[pl.ds(16 * (_L + 1), 16)] = \
            valid * psum / (safe * jnp.float32(_L))
        pltpu.sync_copy(stats_loc, shared_stats.at[g])

    plsc.subcore_barrier()

    @pl.when(g < _G)
    def _():
        pltpu.sync_copy(shared_stats.at[g ^ 1], other_loc)
        v_own = stats_loc[pl.ds(16 * _L, 16)]
        v_oth = other_loc[pl.ds(16 * _L, 16)]
        t_own = [stats_loc[pl.ds(16 * l, 16)] for l in range(_L)]
        t_oth = [other_loc[pl.ds(16 * l, 16)] for l in range(_L)]

        def _push(i, acc):
            iv = zi + i
            vi = plsc.load_gather(stats_loc, [iv + 16 * _L])
            e_own = zf
            e_oth = zf
            for l in range(_L):
                ti = plsc.load_gather(stats_loc, [iv + 16 * l])
                d0 = t_own[l] - ti
                d1 = t_oth[l] - ti
                e_own = e_own + jnp.exp(-(d0 * d0))
                e_oth = e_oth + jnp.exp(-(d1 * d1))
            return acc + vi * (v_own * e_own + v_oth * e_oth)

        pacc_buf[pl.ds(0, 16)] = lax.fori_loop(0, 16, _push, zf)
        pltpu.sync_copy(pacc_buf, shared_push.at[g])

    plsc.subcore_barrier()

    @pl.when(g == 0)
    def _():
        pltpu.sync_copy(shared_stats, all_stats)
        pltpu.sync_copy(shared_push, all_push)
        total_pull = zf
        total_push = zf
        for b in range(_B):
            g0, g1 = _GPB * b, _GPB * b + 1
            va = all_stats[g0, pl.ds(16 * _L, 16)]
            vb = all_stats[g1, pl.ds(16 * _L, 16)]
            nn = zf + (jnp.sum(va) + jnp.sum(vb))
            pull_img = (zf + (jnp.sum(all_stats[g0, pl.ds(16 * (_L + 1), 16)])
                              + jnp.sum(all_stats[g1, pl.ds(16 * (_L + 1), 16)]))) \
                / (nn + jnp.float32(_EPS))
            push_sum = zf + (jnp.sum(all_push[g0, pl.ds(0, 16)])
                             + jnp.sum(all_push[g1, pl.ds(0, 16)]))
            push_img = jnp.where(
                nn > 1.0,
                push_sum / ((nn - 1.0) * nn + jnp.float32(_EPS)),
                zf)
            total_pull = total_pull + pull_img
            total_push = total_push + push_img
        out_v[...] = (jnp.where(lane == 0, total_pull, zf)
                      + jnp.where(lane == 1, total_push, zf))
        pltpu.sync_copy(out_v, out_hbm)


@functools.partial(
    pl.kernel,
    out_type=jax.ShapeDtypeStruct((16,), jnp.float32),
    mesh=plsc.VectorSubcoreMesh(core_axis_name="c", subcore_axis_name="s",
                                num_cores=1),
    compiler_params=pltpu.CompilerParams(needs_layout_passes=False),
    scratch_types=[
        pltpu.VMEM((_K * 2 * 16,), jnp.int32),
        pltpu.VMEM((_NCH, 128), jnp.int32),
        pltpu.VMEM((_NCH, 128), jnp.float32),
        pltpu.VMEM((_K, 16), jnp.float32),
        pltpu.VMEM((128,), jnp.float32),
        pltpu.VMEM((128,), jnp.float32),
        pltpu.VMEM((128,), jnp.float32),
        pltpu.VMEM((_G, 128), jnp.float32),
        pltpu.VMEM((_G, 128), jnp.float32),
        pltpu.VMEM((16,), jnp.float32),
        pltpu.VMEM_SHARED((_G, 128), jnp.float32),
        pltpu.VMEM_SHARED((_G, 128), jnp.float32),
        pltpu.SemaphoreType.DMA,
    ],
)
def _ae_loss_sc(tags_hbm, kpi_hbm, out_hbm, *scratch):
    _sc_body(tags_hbm, kpi_hbm, out_hbm, *scratch)


@jax.jit
def kernel(tags, keypoint_indices):
    tags_flat = (tags
                 .reshape(_B, _L * _K, _H // 8, 8, _W // 128, 128)
                 .transpose(0, 1, 2, 4, 3, 5)
                 .reshape(-1))
    kpi_flat = keypoint_indices.astype(jnp.int32).reshape(-1)
    out = _ae_loss_sc(tags_flat, kpi_flat)
    pull_loss = out[0] * jnp.float32(_LOSS_WEIGHT)
    push_loss = out[1] * jnp.float32(_LOSS_WEIGHT * _PUSH_FACTOR)
    return (pull_loss, push_loss)


# speedup vs baseline: 13.8915x; 1.0191x over previous
import functools

import jax
import jax.numpy as jnp
from jax import lax
from jax.experimental import pallas as pl
from jax.experimental.pallas import tpu as pltpu
from jax.experimental.pallas import tpu_sc as plsc

_B, _N, _K, _L, _H, _W = 4, 32, 17, 4, 256, 256
_HW = _H * _W
_I = _B * _N
_G = _I // 16
_GPB = _N // 16
_EPG = _K * _L * 16
_NCH = 9
_LOSS_WEIGHT = 1.0
_PUSH_FACTOR = 0.5
_EPS = 1e-6


def _sc_body(tags_hbm, kpi_hbm, out_hbm,
             kpi_loc, idx_loc, kt_loc, m_loc, stats_loc, other_loc,
             pacc_buf, all_loc, out_v, shared_all, sem):
    g = lax.axis_index("s")
    zf = jnp.zeros((16,), jnp.float32)
    zi = jnp.zeros((16,), jnp.int32)
    lane = lax.iota(jnp.int32, 16)

    @pl.when(g < _G)
    def _():
        pltpu.sync_copy(kpi_hbm.at[pl.ds(g * (_K * 2 * 16), _K * 2 * 16)],
                        kpi_loc)
        b = g // _GPB

        for c in range(4):
            idx_loc[_NCH - 1, pl.ds(64 + 16 * c, 16)] = zi

        for k in [_K - 1] + list(range(_K - 1)):
            base = lane * (_K * 2) + 2 * k
            idxv = plsc.load_gather(kpi_loc, [base])
            mv = plsc.load_gather(kpi_loc, [base + 1])
            m_loc[k, :] = jnp.where(mv != 0, jnp.float32(1.0),
                                    jnp.float32(0.0))
            pv = (((idxv >> 11) << 11)
                  + (((idxv >> 7) & 1) << 10)
                  + (((idxv >> 8) & 7) << 7)
                  + (idxv & 127))
            coff = (b * _L) * _K * _HW + k * _HW
            for l in range(_L):
                e = 4 * k + l
                idx_loc[e >> 3, pl.ds((e & 7) * 16, 16)] = \
                    pv + (coff + l * _K * _HW)
            if k == _K - 1:
                c = _NCH - 1
                pltpu.make_async_copy(
                    tags_hbm.at[idx_loc.at[c]], kt_loc.at[c], sem).start()
            elif k % 2 == 1:
                c = (4 * k + 3) // 8
                pltpu.make_async_copy(
                    tags_hbm.at[idx_loc.at[c]], kt_loc.at[c], sem).start()
        for c in range(_NCH):
            pltpu.make_async_copy(
                tags_hbm.at[idx_loc.at[c]], kt_loc.at[c], sem).wait()

        def _kt(k, l):
            e = 4 * k + l
            return kt_loc[e >> 3, pl.ds((e & 7) * 16, 16)]

        def _sums(k, carry):
            cnt, t0, t1, t2, t3 = carry
            mv = m_loc[k, :]
            return (cnt + mv,
                    t0 + mv * _kt(k, 0), t1 + mv * _kt(k, 1),
                    t2 + mv * _kt(k, 2), t3 + mv * _kt(k, 3))

        cnt, t0, t1, t2, t3 = lax.fori_loop(
            0, _K, _sums, (zf, zf, zf, zf, zf))
        safe = jnp.maximum(cnt, 1.0)
        tg = [t0 / safe, t1 / safe, t2 / safe, t3 / safe]
        valid = jnp.where(cnt > 0.0, jnp.float32(1.0), jnp.float32(0.0))

        def _pull(k, acc):
            mv = m_loc[k, :]
            s = zf
            for l in range(_L):
                d = _kt(k, l) - tg[l]
                s = s + d * d
            return acc + mv * s

        psum = lax.fori_loop(0, _K, _pull, zf)
        for l in range(_L):
            stats_loc[pl.ds(16 * l, 16)] = tg[l]
        stats_loc[pl.ds(16 * _L, 16)] = valid
        stats_loc[pl.ds(16 * (_L + 1), 16)] = \
            valid * psum / (safe * jnp.float32(_L))
        pltpu.sync_copy(stats_loc, shared_all.at[g])

    plsc.subcore_barrier()

    @pl.when(g < _G)
    def _():
        pltpu.sync_copy(shared_all.at[g ^ 1], other_loc)
        v_own = stats_loc[pl.ds(16 * _L, 16)]
        v_oth = other_loc[pl.ds(16 * _L, 16)]
        t_own = [stats_loc[pl.ds(16 * l, 16)] for l in range(_L)]
        t_oth = [other_loc[pl.ds(16 * l, 16)] for l in range(_L)]

        def _push(i, acc):
            iv = zi + i
            vi = plsc.load_gather(stats_loc, [iv + 16 * _L])
            e_own = zf
            e_oth = zf
            for l in range(_L):
                ti = plsc.load_gather(stats_loc, [iv + 16 * l])
                d0 = t_own[l] - ti
                d1 = t_oth[l] - ti
                e_own = e_own + jnp.exp(-(d0 * d0))
                e_oth = e_oth + jnp.exp(-(d1 * d1))
            return acc + vi * (v_own * e_own + v_oth * e_oth)

        pacc_buf[pl.ds(0, 16)] = lax.fori_loop(0, 16, _push, zf)
        pltpu.sync_copy(pacc_buf, shared_all.at[g + _G])

    plsc.subcore_barrier()

    @pl.when(g == 0)
    def _():
        pltpu.sync_copy(shared_all, all_loc)
        total_pull = zf
        total_push = zf
        for b in range(_B):
            g0, g1 = _GPB * b, _GPB * b + 1
            va = all_loc[g0, pl.ds(16 * _L, 16)]
            vb = all_loc[g1, pl.ds(16 * _L, 16)]
            nn = zf + (jnp.sum(va) + jnp.sum(vb))
            pull_img = (zf + (jnp.sum(all_loc[g0, pl.ds(16 * (_L + 1), 16)])
                              + jnp.sum(all_loc[g1, pl.ds(16 * (_L + 1), 16)]))) \
                / (nn + jnp.float32(_EPS))
            push_sum = zf + (jnp.sum(all_loc[_G + g0, pl.ds(0, 16)])
                             + jnp.sum(all_loc[_G + g1, pl.ds(0, 16)]))
            push_img = jnp.where(
                nn > 1.0,
                push_sum / ((nn - 1.0) * nn + jnp.float32(_EPS)),
                zf)
            total_pull = total_pull + pull_img
            total_push = total_push + push_img
        out_v[...] = (jnp.where(lane == 0, total_pull, zf)
                      + jnp.where(lane == 1, total_push, zf))
        pltpu.sync_copy(out_v, out_hbm)


@functools.partial(
    pl.kernel,
    out_type=jax.ShapeDtypeStruct((16,), jnp.float32),
    mesh=plsc.VectorSubcoreMesh(core_axis_name="c", subcore_axis_name="s",
                                num_cores=1),
    compiler_params=pltpu.CompilerParams(needs_layout_passes=False),
    scratch_types=[
        pltpu.VMEM((_K * 2 * 16,), jnp.int32),
        pltpu.VMEM((_NCH, 128), jnp.int32),
        pltpu.VMEM((_NCH, 128), jnp.float32),
        pltpu.VMEM((_K, 16), jnp.float32),
        pltpu.VMEM((128,), jnp.float32),
        pltpu.VMEM((128,), jnp.float32),
        pltpu.VMEM((128,), jnp.float32),
        pltpu.VMEM((2 * _G, 128), jnp.float32),
        pltpu.VMEM((16,), jnp.float32),
        pltpu.VMEM_SHARED((2 * _G, 128), jnp.float32),
        pltpu.SemaphoreType.DMA,
    ],
)
def _ae_loss_sc(tags_hbm, kpi_hbm, out_hbm, *scratch):
    _sc_body(tags_hbm, kpi_hbm, out_hbm, *scratch)


@jax.jit
def kernel(tags, keypoint_indices):
    tags_flat = (tags
                 .reshape(_B, _L * _K, _H // 8, 8, _W // 128, 128)
                 .transpose(0, 1, 2, 4, 3, 5)
                 .reshape(-1))
    kpi_flat = keypoint_indices.astype(jnp.int32).reshape(-1)
    out = _ae_loss_sc(tags_flat, kpi_flat)
    pull_loss = out[0] * jnp.float32(_LOSS_WEIGHT)
    push_loss = out[1] * jnp.float32(_LOSS_WEIGHT * _PUSH_FACTOR)
    return (pull_loss, push_loss)


# speedup vs baseline: 13.9236x; 1.0023x over previous
import functools

import jax
import jax.numpy as jnp
from jax import lax
from jax.experimental import pallas as pl
from jax.experimental.pallas import tpu as pltpu
from jax.experimental.pallas import tpu_sc as plsc

_B, _N, _K, _L, _H, _W = 4, 32, 17, 4, 256, 256
_HW = _H * _W
_I = _B * _N
_G = _I // 16
_GPB = _N // 16
_EPG = _K * _L * 16
_NCH = 9
_LOSS_WEIGHT = 1.0
_PUSH_FACTOR = 0.5
_EPS = 1e-6


def _sc_body(tags_hbm, kpi_hbm, out_hbm,
             kpi_loc, idx_loc, kt_loc, m_loc, stats_loc, other_loc,
             pacc_buf, all_loc, out_v, shared_all, sem):
    g = lax.axis_index("s")
    zf = jnp.zeros((16,), jnp.float32)
    zi = jnp.zeros((16,), jnp.int32)
    lane = lax.iota(jnp.int32, 16)

    @pl.when(g < _G)
    def _():
        pltpu.sync_copy(kpi_hbm.at[pl.ds(g * (_K * 2 * 16), _K * 2 * 16)],
                        kpi_loc)
        b = g // _GPB

        for c in range(4):
            idx_loc[_NCH - 1, pl.ds(64 + 16 * c, 16)] = zi

        for k in [_K - 1] + list(range(_K - 1)):
            base = lane * (_K * 2) + 2 * k
            idxv = plsc.load_gather(kpi_loc, [base])
            mv = plsc.load_gather(kpi_loc, [base + 1])
            m_loc[k, :] = jnp.where(mv != 0, jnp.float32(1.0),
                                    jnp.float32(0.0))
            pv = (((idxv >> 11) << 11)
                  + (((idxv >> 7) & 1) << 10)
                  + (((idxv >> 8) & 7) << 7)
                  + (idxv & 127))
            coff = (b * _L) * _K * _HW + k * _HW
            for l in range(_L):
                e = 4 * k + l
                idx_loc[e >> 3, pl.ds((e & 7) * 16, 16)] = \
                    pv + (coff + l * _K * _HW)
            if k == _K - 1:
                c = _NCH - 1
                pltpu.make_async_copy(
                    tags_hbm.at[idx_loc.at[c]], kt_loc.at[c], sem).start()
            elif k % 2 == 1:
                c = (4 * k + 3) // 8
                pltpu.make_async_copy(
                    tags_hbm.at[idx_loc.at[c]], kt_loc.at[c], sem).start()
        for c in range(_NCH):
            pltpu.make_async_copy(
                tags_hbm.at[idx_loc.at[c]], kt_loc.at[c], sem).wait()

        def _kt(k, l):
            e = 4 * k + l
            return kt_loc[e >> 3, pl.ds((e & 7) * 16, 16)]

        def _sums(k, carry):
            cnt, t0, t1, t2, t3 = carry
            mv = m_loc[k, :]
            return (cnt + mv,
                    t0 + mv * _kt(k, 0), t1 + mv * _kt(k, 1),
                    t2 + mv * _kt(k, 2), t3 + mv * _kt(k, 3))

        cnt, t0, t1, t2, t3 = lax.fori_loop(
            0, _K, _sums, (zf, zf, zf, zf, zf))
        safe = jnp.maximum(cnt, 1.0)
        tg = [t0 / safe, t1 / safe, t2 / safe, t3 / safe]
        valid = jnp.where(cnt > 0.0, jnp.float32(1.0), jnp.float32(0.0))

        def _pull(k, acc):
            mv = m_loc[k, :]
            s = zf
            for l in range(_L):
                d = _kt(k, l) - tg[l]
                s = s + d * d
            return acc + mv * s

        psum = lax.fori_loop(0, _K, _pull, zf)
        for l in range(_L):
            stats_loc[pl.ds(16 * l, 16)] = tg[l]
        stats_loc[pl.ds(16 * _L, 16)] = valid
        stats_loc[pl.ds(16 * (_L + 1), 16)] = \
            valid * psum / (safe * jnp.float32(_L))
        pltpu.sync_copy(stats_loc, shared_all.at[g])

    plsc.subcore_barrier()

    @pl.when(g < _G)
    def _():
        pltpu.sync_copy(shared_all.at[g ^ 1], other_loc)
        v_own = stats_loc[pl.ds(16 * _L, 16)]
        v_oth = other_loc[pl.ds(16 * _L, 16)]
        t_own = [stats_loc[pl.ds(16 * l, 16)] for l in range(_L)]
        t_oth = [other_loc[pl.ds(16 * l, 16)] for l in range(_L)]

        def _push(i, acc):
            iv = zi + i
            vi = plsc.load_gather(stats_loc, [iv + 16 * _L])
            e_own = zf
            e_oth = zf
            for l in range(_L):
                ti = plsc.load_gather(stats_loc, [iv + 16 * l])
                d0 = t_own[l] - ti
                d1 = t_oth[l] - ti
                e_own = e_own + jnp.exp(-(d0 * d0))
                e_oth = e_oth + jnp.exp(-(d1 * d1))
            return acc + vi * (v_own * e_own + v_oth * e_oth)

        pacc_buf[pl.ds(0, 16)] = lax.fori_loop(0, 16, _push, zf)
        pltpu.sync_copy(pacc_buf, shared_all.at[g + _G])

    plsc.subcore_barrier()

    @pl.when(g == 0)
    def _():
        pltpu.sync_copy(shared_all, all_loc)
        total_pull = zf
        total_push = zf
        for b in range(_B):
            g0, g1 = _GPB * b, _GPB * b + 1
            va = all_loc[g0, pl.ds(16 * _L, 16)]
            vb = all_loc[g1, pl.ds(16 * _L, 16)]
            nn = zf + (jnp.sum(va) + jnp.sum(vb))
            pull_img = (zf + (jnp.sum(all_loc[g0, pl.ds(16 * (_L + 1), 16)])
                              + jnp.sum(all_loc[g1, pl.ds(16 * (_L + 1), 16)]))) \
                / (nn + jnp.float32(_EPS))
            push_sum = zf + (jnp.sum(all_loc[_G + g0, pl.ds(0, 16)])
                             + jnp.sum(all_loc[_G + g1, pl.ds(0, 16)]))
            push_img = jnp.where(
                nn > 1.0,
                push_sum / ((nn - 1.0) * nn + jnp.float32(_EPS)),
                zf)
            total_pull = total_pull + pull_img
            total_push = total_push + push_img
        out_v[...] = (jnp.where(lane == 0, total_pull, zf)
                      + jnp.where(lane == 1, total_push, zf))
        pltpu.sync_copy(out_v, out_hbm)


@functools.partial(
    pl.kernel,
    out_type=jax.ShapeDtypeStruct((16,), jnp.float32),
    mesh=plsc.VectorSubcoreMesh(core_axis_name="c", subcore_axis_name="s",
                                num_cores=1),
    compiler_params=pltpu.CompilerParams(needs_layout_passes=False, skip_device_barrier=True, disable_bounds_checks=True, disable_semaphore_checks=True),
    scratch_types=[
        pltpu.VMEM((_K * 2 * 16,), jnp.int32),
        pltpu.VMEM((_NCH, 128), jnp.int32),
        pltpu.VMEM((_NCH, 128), jnp.float32),
        pltpu.VMEM((_K, 16), jnp.float32),
        pltpu.VMEM((128,), jnp.float32),
        pltpu.VMEM((128,), jnp.float32),
        pltpu.VMEM((128,), jnp.float32),
        pltpu.VMEM((2 * _G, 128), jnp.float32),
        pltpu.VMEM((16,), jnp.float32),
        pltpu.VMEM_SHARED((2 * _G, 128), jnp.float32),
        pltpu.SemaphoreType.DMA,
    ],
)
def _ae_loss_sc(tags_hbm, kpi_hbm, out_hbm, *scratch):
    _sc_body(tags_hbm, kpi_hbm, out_hbm, *scratch)


@jax.jit
def kernel(tags, keypoint_indices):
    tags_flat = (tags
                 .reshape(_B, _L * _K, _H // 8, 8, _W // 128, 128)
                 .transpose(0, 1, 2, 4, 3, 5)
                 .reshape(-1))
    kpi_flat = keypoint_indices.astype(jnp.int32).reshape(-1)
    out = _ae_loss_sc(tags_flat, kpi_flat)
    pull_loss = out[0] * jnp.float32(_LOSS_WEIGHT)
    push_loss = out[1] * jnp.float32(_LOSS_WEIGHT * _PUSH_FACTOR)
    return (pull_loss, push_loss)
